# Initial kernel scaffold; baseline (speedup 1.0000x reference)
import jax
import jax.numpy as jnp
from jax.experimental import pallas as pl


def kernel(hidden_states, W_gate):
    raise NotImplementedError("write your pallas kernel here")


# speedup vs baseline: 1.0307x; 1.0307x over previous
import functools

import jax
import jax.numpy as jnp
from jax.experimental import pallas as pl

NUM_EXPERTS = 64
TOP_K = 8
BT = 512


def _router_body(x_ref, w_ref, logits_ref, idx_ref, wts_ref):
    x = x_ref[...]
    w = w_ref[...]
    logits = jax.lax.dot_general(
        x, w, (((1,), (1,)), ((), ())), preferred_element_type=jnp.float32
    )
    logits_ref[...] = logits

    col = jax.lax.broadcasted_iota(jnp.int32, (BT, NUM_EXPERTS), 1)
    neg = jnp.float32(-3.4e38)
    cur = logits
    idxs = []
    vals = []
    for _ in range(TOP_K):
        m = jnp.max(cur, axis=1, keepdims=True)
        i = jnp.min(jnp.where(cur == m, col, NUM_EXPERTS), axis=1, keepdims=True)
        idxs.append(i)
        vals.append(m)
        cur = jnp.where(col == i, neg, cur)
    idx = jnp.concatenate(idxs, axis=1)
    vals = jnp.concatenate(vals, axis=1)
    e = jnp.exp(vals - vals[:, 0:1])
    wts = e / jnp.sum(e, axis=1, keepdims=True)
    idx_ref[...] = idx
    wts_ref[...] = wts


@functools.partial(jax.jit, static_argnames=())
def kernel(hidden_states, W_gate):
    if hidden_states.ndim == 3:
        hidden_states = hidden_states.reshape(-1, hidden_states.shape[-1])
    T, H = hidden_states.shape
    E = W_gate.shape[0]
    grid = (T // BT,)
    logits, idx, wts = pl.pallas_call(
        _router_body,
        grid=grid,
        in_specs=[
            pl.BlockSpec((BT, H), lambda i: (i, 0)),
            pl.BlockSpec((E, H), lambda i: (0, 0)),
        ],
        out_specs=[
            pl.BlockSpec((BT, E), lambda i: (i, 0)),
            pl.BlockSpec((BT, TOP_K), lambda i: (i, 0)),
            pl.BlockSpec((BT, TOP_K), lambda i: (i, 0)),
        ],
        out_shape=[
            jax.ShapeDtypeStruct((T, E), jnp.float32),
            jax.ShapeDtypeStruct((T, TOP_K), jnp.int32),
            jax.ShapeDtypeStruct((T, TOP_K), jnp.float32),
        ],
    )(hidden_states, W_gate)
    return (logits, idx, wts)


# speedup vs baseline: 1.2827x; 1.2445x over previous
import jax
import jax.numpy as jnp
from jax.experimental import pallas as pl

NUM_EXPERTS = 64
TOP_K = 8
BT = 512

def _m(v):
    return jnp.where(v >= 0, v, jnp.int32(-(2**31)) - v)


def _router_body(x_ref, w_ref, logits_ref, idx_ref, wts_ref):
    x = x_ref[...]
    w = w_ref[...]
    logits = jax.lax.dot_general(
        x, w, (((1,), (1,)), ((), ())), preferred_element_type=jnp.float32
    )
    logits_ref[...] = logits

    col = jax.lax.broadcasted_iota(jnp.int32, (BT, NUM_EXPERTS), 1)
    b = jax.lax.bitcast_convert_type(logits, jnp.int32)
    key = (_m(b) & jnp.int32(~63)) | (jnp.int32(NUM_EXPERTS - 1) - col)
    cur = jax.lax.bitcast_convert_type(_m(key), jnp.float32)

    neg_inf = jnp.float32(-jnp.inf)
    ms = []
    for _ in range(TOP_K):
        m = jnp.max(cur, axis=1, keepdims=True)
        ms.append(m)
        cur = jnp.where(cur == m, neg_inf, cur)
    fm = jnp.concatenate(ms, axis=1)

    kk = _m(jax.lax.bitcast_convert_type(fm, jnp.int32))
    idx_ref[...] = jnp.int32(NUM_EXPERTS - 1) - (kk & jnp.int32(63))
    e = jnp.exp(fm - fm[:, 0:1])
    wts_ref[...] = e / jnp.sum(e, axis=1, keepdims=True)


def kernel(hidden_states, W_gate):
    if hidden_states.ndim == 3:
        hidden_states = hidden_states.reshape(-1, hidden_states.shape[-1])
    T, H = hidden_states.shape
    E = W_gate.shape[0]
    grid = (T // BT,)
    logits, idx, wts = pl.pallas_call(
        _router_body,
        grid=grid,
        in_specs=[
            pl.BlockSpec((BT, H), lambda i: (i, 0)),
            pl.BlockSpec((E, H), lambda i: (0, 0)),
        ],
        out_specs=[
            pl.BlockSpec((BT, E), lambda i: (i, 0)),
            pl.BlockSpec((BT, TOP_K), lambda i: (i, 0)),
            pl.BlockSpec((BT, TOP_K), lambda i: (i, 0)),
        ],
        out_shape=[
            jax.ShapeDtypeStruct((T, E), jnp.float32),
            jax.ShapeDtypeStruct((T, TOP_K), jnp.int32),
            jax.ShapeDtypeStruct((T, TOP_K), jnp.float32),
        ],
    )(hidden_states, W_gate)
    return (logits, idx, wts)
